# baseline (speedup 1.0000x reference)
import functools

import jax
import jax.numpy as jnp
from jax import lax
from jax.experimental import pallas as pl
from jax.experimental.pallas import tpu as pltpu
from jax.experimental.pallas import tpu_sc as plsc


@functools.lru_cache(maxsize=None)
def _build_gather(B, D, NW, NCHUNK, C, NBUF):
    mesh = plsc.VectorSubcoreMesh(core_axis_name="c", subcore_axis_name="s")

    scratch = [pltpu.VMEM((NCHUNK * C,), jnp.int32)]
    scratch += [pltpu.VMEM((C, D), jnp.float32)] * NBUF
    scratch += [pltpu.SemaphoreType.DMA] * (2 * NBUF)

    @functools.partial(
        pl.kernel,
        mesh=mesh,
        out_type=jax.ShapeDtypeStruct((B, D), jnp.float32),
        scratch_types=scratch,
    )
    def k(idx_hbm, table_hbm, out_hbm, idx_v, *rest):
        bufs = rest[:NBUF]
        gsems = rest[NBUF:2 * NBUF]
        ssems = rest[2 * NBUF:]
        wid = lax.axis_index("s") * 2 + lax.axis_index("c")
        base = wid * (NCHUNK * C)
        pltpu.sync_copy(idx_hbm.at[pl.ds(base, NCHUNK * C)], idx_v)

        gh = [None] * NBUF
        sh = [None] * NBUF
        for j in range(min(NBUF - 1, NCHUNK)):
            gh[j] = pltpu.async_copy(
                table_hbm.at[idx_v.at[pl.ds(j * C, C)]], bufs[j], gsems[j])
        for i in range(NCHUNK):
            cur = i % NBUF
            pre = i + NBUF - 1
            if pre < NCHUNK:
                b = pre % NBUF
                if sh[b] is not None:
                    sh[b].wait()
                    sh[b] = None
                gh[b] = pltpu.async_copy(
                    table_hbm.at[idx_v.at[pl.ds(pre * C, C)]], bufs[b], gsems[b])
            gh[cur].wait()
            sh[cur] = pltpu.async_copy(
                bufs[cur], out_hbm.at[pl.ds(base + i * C, C)], ssems[cur])
        for h in sh:
            if h is not None:
                h.wait()

    return k


def kernel(class_attributes, class_idx, seen_class_embeddings, W1, b1, W2, b2, Wp, bp):
    B = class_idx.shape[0]
    D = seen_class_embeddings.shape[1]
    NW = 32
    C = 32
    NBUF = 3
    NCHUNK = B // (NW * C)
    idx = jnp.asarray(class_idx, jnp.int32)
    return _build_gather(B, D, NW, NCHUNK, C, NBUF)(idx, seen_class_embeddings)


# speedup vs baseline: 1.0098x; 1.0098x over previous
import functools

import jax
import jax.numpy as jnp
from jax import lax
from jax.experimental import pallas as pl
from jax.experimental.pallas import tpu as pltpu
from jax.experimental.pallas import tpu_sc as plsc


@functools.lru_cache(maxsize=None)
def _build_gather(B, D, NW, NCHUNK, C, NBUF):
    mesh = plsc.VectorSubcoreMesh(core_axis_name="c", subcore_axis_name="s")

    scratch = [pltpu.VMEM((NCHUNK * C,), jnp.int32)]
    scratch += [pltpu.VMEM((C, D), jnp.float32)] * NBUF
    scratch += [pltpu.SemaphoreType.DMA] * (2 * NBUF)

    @functools.partial(
        pl.kernel,
        mesh=mesh,
        out_type=jax.ShapeDtypeStruct((B, D), jnp.float32),
        scratch_types=scratch,
    )
    def k(idx_hbm, table_hbm, out_hbm, idx_v, *rest):
        bufs = rest[:NBUF]
        gsems = rest[NBUF:2 * NBUF]
        ssems = rest[2 * NBUF:]
        wid = lax.axis_index("s") * 2 + lax.axis_index("c")
        base = wid * (NCHUNK * C)
        pltpu.sync_copy(idx_hbm.at[pl.ds(base, NCHUNK * C)], idx_v)

        gh = [None] * NBUF
        sh = [None] * NBUF
        for j in range(min(NBUF - 1, NCHUNK)):
            gh[j] = pltpu.async_copy(
                table_hbm.at[idx_v.at[pl.ds(j * C, C)]], bufs[j], gsems[j])
        for i in range(NCHUNK):
            cur = i % NBUF
            pre = i + NBUF - 1
            if pre < NCHUNK:
                b = pre % NBUF
                if sh[b] is not None:
                    sh[b].wait()
                    sh[b] = None
                gh[b] = pltpu.async_copy(
                    table_hbm.at[idx_v.at[pl.ds(pre * C, C)]], bufs[b], gsems[b])
            gh[cur].wait()
            sh[cur] = pltpu.async_copy(
                bufs[cur], out_hbm.at[pl.ds(base + i * C, C)], ssems[cur])
        for h in sh:
            if h is not None:
                h.wait()

    return k


def kernel(class_attributes, class_idx, seen_class_embeddings, W1, b1, W2, b2, Wp, bp):
    B = class_idx.shape[0]
    D = seen_class_embeddings.shape[1]
    NW = 32
    C = 16
    NBUF = 6
    NCHUNK = B // (NW * C)
    idx = jnp.asarray(class_idx, jnp.int32)
    return _build_gather(B, D, NW, NCHUNK, C, NBUF)(idx, seen_class_embeddings)
